# baseline (speedup 1.0000x reference)
import functools

import jax
import jax.numpy as jnp
from jax import lax
from jax.experimental import pallas as pl
from jax.experimental.pallas import tpu as pltpu
from jax.experimental.pallas import tpu_sc as plsc

_N_FIELDS = 26
_VOCAB = 100000
_EMB_DIM = 32
_BATCH = 16384
_L = 16

_CH = 256


@functools.cache
def _build_sc_kernel():
    info = plsc.get_sparse_core_info()
    nc, ns = info.num_cores, info.num_subcores
    nw = nc * ns
    bpw = _BATCH // nw
    steps = bpw // _CH

    mesh = plsc.VectorSubcoreMesh(core_axis_name="c", subcore_axis_name="s")

    @functools.partial(
        pl.kernel,
        mesh=mesh,
        out_type=jax.ShapeDtypeStruct((_N_FIELDS, _BATCH, _EMB_DIM), jnp.float32),
        scratch_types=[
            pltpu.VMEM((_N_FIELDS, bpw), jnp.int32),
            pltpu.VMEM((_CH,), jnp.int32),
            pltpu.VMEM((_CH,), jnp.int32),
            pltpu.VMEM((_CH, 4 * _EMB_DIM), jnp.float32),
            pltpu.VMEM((_CH, _EMB_DIM), jnp.float32),
            pltpu.SemaphoreType.DMA,
            pltpu.SemaphoreType.DMA,
        ],
        compiler_params=pltpu.CompilerParams(
            use_tc_tiling_on_sc=True, needs_layout_passes=False
        ),
    )
    def sc_embed(idx_hbm, tab_hbm, out_hbm, idx_v, rid_v, cb_v, raw_v, rows_v,
                 gsem, wsem):
        wid = lax.axis_index("s") * nc + lax.axis_index("c")
        b0 = wid * bpw
        lanes = lax.iota(jnp.int32, _L)

        pltpu.sync_copy(idx_hbm.at[:, pl.ds(b0, bpw)], idx_v)

        def chunk(t, carry):
            i, s = t // steps, t % steps

            def prep(k, c):
                v = idx_v[i, pl.ds(s * _CH + k * _L, _L)]
                rid_v[pl.ds(k * _L, _L)] = i * (_VOCAB // 4) + (v >> 2)
                cb_v[pl.ds(k * _L, _L)] = (v & 3) * _EMB_DIM
                return c

            lax.fori_loop(0, _CH // _L, prep, 0)
            pltpu.async_copy(tab_hbm.at[rid_v], raw_v, gsem).wait()

            def extract(k, c):
                rvec = lanes + k * _L
                cb = cb_v[pl.ds(k * _L, _L)]

                def comp(e, c2):
                    w = plsc.load_gather(raw_v, [rvec, cb + e])
                    plsc.store_scatter(
                        rows_v, [rvec, jnp.full((_L,), e, jnp.int32)], w
                    )
                    return c2

                lax.fori_loop(0, _EMB_DIM, comp, 0)
                return c

            lax.fori_loop(0, _CH // _L, extract, 0)
            pltpu.async_copy(
                rows_v, out_hbm.at[i, pl.ds(b0 + s * _CH, _CH), :], wsem
            ).wait()
            return carry

        lax.fori_loop(0, _N_FIELDS * steps, chunk, 0)

    return sc_embed


def kernel(sparse_inputs, tables):
    idx_t = sparse_inputs.astype(jnp.int32).T
    tab128 = tables.reshape(_N_FIELDS * _VOCAB // 4, 4 * _EMB_DIM)
    out3 = _build_sc_kernel()(idx_t, tab128)
    return out3.transpose(1, 0, 2).reshape(_BATCH, _N_FIELDS * _EMB_DIM)
